# Initial kernel scaffold; baseline (speedup 1.0000x reference)
import jax
import jax.numpy as jnp
from jax.experimental import pallas as pl


def kernel(Q, K, reference_angles, probes, q_weights_raw, q_bias):
    raise NotImplementedError("write your pallas kernel here")


# speedup vs baseline: 3.1879x; 3.1879x over previous
import functools

import jax
import jax.numpy as jnp
from jax import lax
from jax.experimental import pallas as pl

NUM_BINS = 128
HEAD_DIM = 128
NUM_FREQS = HEAD_DIM // 2
NUM_QUERIES = 4096
NUM_KEYS = 32768
EPS = 1e-08

GRID = 16
KBLK = NUM_KEYS // GRID
QBLK = NUM_QUERIES // GRID


def _fused_kernel(k_ref, probes_ref, probes_sw_ref, cos_i_ref, sin_i_ref,
                  qx_ref, qy_ref, pxq_ref, pyq_ref, cosf_ref, sinf_ref,
                  wraw_ref, bias_ref, kout_ref, qout_ref):
    rot_p = probes_ref[...] * cos_i_ref[...] + probes_sw_ref[...] * sin_i_ref[...]
    kout_ref[...] = lax.dot_general(
        k_ref[...], rot_p,
        dimension_numbers=(((1,), (1,)), ((), ())),
        preferred_element_type=jnp.float32,
        precision=lax.Precision.HIGHEST)

    cosf = cosf_ref[...]
    sinf = sinf_ref[...]
    pxq = pxq_ref[...]
    pyq = pyq_ref[...]
    px = pxq * cosf - pyq * sinf
    py = pxq * sinf + pyq * cosf

    wraw = wraw_ref[...]
    w = -(jnp.maximum(wraw, 0.0) + jnp.log1p(jnp.exp(-jnp.abs(wraw))))

    qx = qx_ref[...]
    qy = qy_ref[...]
    acc = jnp.zeros((QBLK, NUM_BINS), dtype=jnp.float32)
    for f in range(NUM_FREQS):
        dx = qx[:, f:f + 1] - px[f:f + 1, :]
        dy = qy[:, f:f + 1] - py[f:f + 1, :]
        d = jnp.sqrt(dx * dx + dy * dy + EPS)
        acc = acc + d * w[f:f + 1, :]
    qout_ref[...] = acc + bias_ref[...]


@functools.partial(jax.jit, static_argnums=())
def kernel(Q, K, reference_angles, probes, q_weights_raw, q_bias):
    cos_f = jnp.cos(reference_angles)
    sin_f = jnp.sin(reference_angles)
    cos_i = jnp.repeat(cos_f, 2).reshape(1, HEAD_DIM)
    sin_i = jnp.stack([-sin_f, sin_f], axis=-1).reshape(1, HEAD_DIM)
    probes_sw = probes.reshape(NUM_BINS, NUM_FREQS, 2)[..., ::-1].reshape(
        NUM_BINS, HEAD_DIM)

    qf = Q.reshape(NUM_QUERIES, NUM_FREQS, 2)
    qx = qf[..., 0]
    qy = qf[..., 1]
    pf = probes.reshape(NUM_BINS, NUM_FREQS, 2)
    pxq = pf[..., 0].T
    pyq = pf[..., 1].T
    cosf_col = cos_f.reshape(NUM_FREQS, 1)
    sinf_col = sin_f.reshape(NUM_FREQS, 1)
    wraw_t = q_weights_raw.T
    bias_row = q_bias.reshape(1, NUM_BINS)

    full = lambda shape: pl.BlockSpec(shape, lambda i: (0, 0))
    kout, qout = pl.pallas_call(
        _fused_kernel,
        grid=(GRID,),
        in_specs=[
            pl.BlockSpec((KBLK, HEAD_DIM), lambda i: (i, 0)),
            full((NUM_BINS, HEAD_DIM)),
            full((NUM_BINS, HEAD_DIM)),
            full((1, HEAD_DIM)),
            full((1, HEAD_DIM)),
            pl.BlockSpec((QBLK, NUM_FREQS), lambda i: (i, 0)),
            pl.BlockSpec((QBLK, NUM_FREQS), lambda i: (i, 0)),
            full((NUM_FREQS, NUM_BINS)),
            full((NUM_FREQS, NUM_BINS)),
            full((NUM_FREQS, 1)),
            full((NUM_FREQS, 1)),
            full((NUM_FREQS, NUM_BINS)),
            full((1, NUM_BINS)),
        ],
        out_specs=[
            pl.BlockSpec((KBLK, NUM_BINS), lambda i: (i, 0)),
            pl.BlockSpec((QBLK, NUM_BINS), lambda i: (i, 0)),
        ],
        out_shape=[
            jax.ShapeDtypeStruct((NUM_KEYS, NUM_BINS), jnp.float32),
            jax.ShapeDtypeStruct((NUM_QUERIES, NUM_BINS), jnp.float32),
        ],
    )(K, probes, probes_sw, cos_i, sin_i, qx, qy, pxq, pyq,
      cosf_col, sinf_col, wraw_t, bias_row)
    return (qout, kout)


# speedup vs baseline: 5.2181x; 1.6368x over previous
import functools

import jax
import jax.numpy as jnp
from jax import lax
from jax.experimental import pallas as pl

NUM_BINS = 128
HEAD_DIM = 128
NUM_FREQS = HEAD_DIM // 2
NUM_QUERIES = 4096
NUM_KEYS = 32768
EPS = 1e-08

GRID = 16
KBLK = NUM_KEYS // GRID
QBLK = NUM_QUERIES // GRID


def _fused_kernel(k_ref, probes_ref, probes_sw_ref, cos_i_ref, sin_i_ref,
                  qxt_ref, qyt_ref, pxq_ref, pyq_ref, cosf_ref, sinf_ref,
                  wraw_ref, bias_ref, kout_ref, qout_ref):
    rot_p = probes_ref[...] * cos_i_ref[...] + probes_sw_ref[...] * sin_i_ref[...]
    kout_ref[...] = lax.dot_general(
        k_ref[...], rot_p,
        dimension_numbers=(((1,), (1,)), ((), ())),
        preferred_element_type=jnp.float32,
        precision=lax.Precision.HIGHEST)

    cosf = cosf_ref[...]
    sinf = sinf_ref[...]
    pxq = pxq_ref[...]
    pyq = pyq_ref[...]
    px = pxq * cosf - pyq * sinf
    py = pxq * sinf + pyq * cosf

    wraw = wraw_ref[...]
    w = -(jnp.maximum(wraw, 0.0) + jnp.log1p(jnp.exp(-jnp.abs(wraw))))

    qxt = qxt_ref[...]
    qyt = qyt_ref[...]
    sq = qxt * qxt + qyt * qyt
    tp = px * px + py * py + EPS
    ones_q = jnp.ones((NUM_FREQS, 1, QBLK), dtype=jnp.float32)
    ones_b = jnp.ones((NUM_FREQS, 1, NUM_BINS), dtype=jnp.float32)
    lhs = jnp.concatenate(
        [qxt[:, None, :], qyt[:, None, :], sq[:, None, :], ones_q], axis=1)
    rhs = jnp.concatenate(
        [(-2.0 * px)[:, None, :], (-2.0 * py)[:, None, :], ones_b,
         tp[:, None, :]], axis=1)
    d2 = lax.dot_general(
        lhs, rhs,
        dimension_numbers=(((1,), (1,)), ((0,), (0,))),
        preferred_element_type=jnp.float32,
        precision=lax.Precision.DEFAULT)
    d2 = jnp.maximum(d2, EPS)
    d = d2 * lax.rsqrt(d2)
    qout_ref[...] = jnp.sum(d * w[:, None, :], axis=0) + bias_ref[...]


@functools.partial(jax.jit, static_argnums=())
def kernel(Q, K, reference_angles, probes, q_weights_raw, q_bias):
    cos_f = jnp.cos(reference_angles)
    sin_f = jnp.sin(reference_angles)
    cos_i = jnp.repeat(cos_f, 2).reshape(1, HEAD_DIM)
    sin_i = jnp.stack([-sin_f, sin_f], axis=-1).reshape(1, HEAD_DIM)
    probes_sw = probes.reshape(NUM_BINS, NUM_FREQS, 2)[..., ::-1].reshape(
        NUM_BINS, HEAD_DIM)

    qf = Q.reshape(NUM_QUERIES, NUM_FREQS, 2)
    qxt = qf[..., 0].T
    qyt = qf[..., 1].T
    pf = probes.reshape(NUM_BINS, NUM_FREQS, 2)
    pxq = pf[..., 0].T
    pyq = pf[..., 1].T
    cosf_col = cos_f.reshape(NUM_FREQS, 1)
    sinf_col = sin_f.reshape(NUM_FREQS, 1)
    wraw_t = q_weights_raw.T
    bias_row = q_bias.reshape(1, NUM_BINS)

    full = lambda shape: pl.BlockSpec(shape, lambda i: (0, 0))
    kout, qout = pl.pallas_call(
        _fused_kernel,
        grid=(GRID,),
        in_specs=[
            pl.BlockSpec((KBLK, HEAD_DIM), lambda i: (i, 0)),
            full((NUM_BINS, HEAD_DIM)),
            full((NUM_BINS, HEAD_DIM)),
            full((1, HEAD_DIM)),
            full((1, HEAD_DIM)),
            pl.BlockSpec((NUM_FREQS, QBLK), lambda i: (0, i)),
            pl.BlockSpec((NUM_FREQS, QBLK), lambda i: (0, i)),
            full((NUM_FREQS, NUM_BINS)),
            full((NUM_FREQS, NUM_BINS)),
            full((NUM_FREQS, 1)),
            full((NUM_FREQS, 1)),
            full((NUM_FREQS, NUM_BINS)),
            full((1, NUM_BINS)),
        ],
        out_specs=[
            pl.BlockSpec((KBLK, NUM_BINS), lambda i: (i, 0)),
            pl.BlockSpec((QBLK, NUM_BINS), lambda i: (i, 0)),
        ],
        out_shape=[
            jax.ShapeDtypeStruct((NUM_KEYS, NUM_BINS), jnp.float32),
            jax.ShapeDtypeStruct((NUM_QUERIES, NUM_BINS), jnp.float32),
        ],
    )(K, probes, probes_sw, cos_i, sin_i, qxt, qyt, pxq, pyq,
      cosf_col, sinf_col, wraw_t, bias_row)
    return (qout, kout)
